# Initial kernel scaffold; baseline (speedup 1.0000x reference)
import functools

import jax
import jax.numpy as jnp
from jax import lax
from jax.experimental import pallas as pl
from jax.experimental.pallas import tpu as pltpu
from jax.experimental.pallas import tpu_sc as plsc

N_NODES = 10000
N_EDGES = 320000
D = 128

NUM_CORES = 2
NUM_SUBCORES = 16
NUM_TILES = NUM_CORES * NUM_SUBCORES
EDGES_PER_TILE = N_EDGES // NUM_TILES
CHUNK = 80
NCHUNKS = EDGES_PER_TILE // CHUNK
ROWS_PER_TILE = N_NODES // NUM_SUBCORES
ZROWS = 125


def _sc_body(feat_h, src_h, dst_h, nout_h, dout_h,
             zbuf, degl, idx_s, idx_d, rows, nacc):
    c = lax.axis_index("c")
    s = lax.axis_index("s")
    wid = c * NUM_SUBCORES + s

    z16 = jnp.zeros((16,), jnp.float32)

    def zero_zbuf(i, carry):
        for k in range(D // 16):
            zbuf[i, pl.ds(k * 16, 16)] = z16
        return carry

    lax.fori_loop(0, ZROWS, zero_zbuf, 0)

    def zero_deg(i, carry):
        degl[pl.ds(i * 16, 16)] = z16
        return carry

    lax.fori_loop(0, N_NODES // 16, zero_deg, 0)

    for q in range(ROWS_PER_TILE // ZROWS):
        pltpu.sync_copy(zbuf, nacc.at[pl.ds(s * ROWS_PER_TILE + q * ZROWS, ZROWS)])
    plsc.subcore_barrier()

    base = wid * EDGES_PER_TILE
    ones16 = jnp.ones((16,), jnp.float32)

    def step(j, carry):
        off = base + j * CHUNK
        pltpu.sync_copy(src_h.at[pl.ds(off, CHUNK)], idx_s)
        pltpu.sync_copy(dst_h.at[pl.ds(off, CHUNK)], idx_d)
        pltpu.sync_copy(feat_h.at[idx_s], rows)
        pltpu.sync_copy(rows, nacc.at[idx_d], add=True)
        for k in range(CHUNK // 16):
            i16 = idx_d[pl.ds(k * 16, 16)]
            plsc.addupdate_scatter(degl, [i16], ones16)
        return carry

    lax.fori_loop(0, NCHUNKS, step, 0)
    plsc.subcore_barrier()

    pltpu.sync_copy(nacc.at[pl.ds(s * ROWS_PER_TILE, ROWS_PER_TILE)],
                    nout_h.at[c, pl.ds(s * ROWS_PER_TILE, ROWS_PER_TILE)])
    pltpu.sync_copy(degl, dout_h.at[wid])


def _sc_aggregate(feature, src, dst):
    mesh = plsc.VectorSubcoreMesh(core_axis_name="c", subcore_axis_name="s")
    f = pl.kernel(
        _sc_body,
        out_type=[
            jax.ShapeDtypeStruct((NUM_CORES, N_NODES, D), jnp.float32),
            jax.ShapeDtypeStruct((NUM_TILES, N_NODES), jnp.float32),
        ],
        mesh=mesh,
        scratch_types=[
            pltpu.VMEM((ZROWS, D), jnp.float32),
            pltpu.VMEM((N_NODES,), jnp.float32),
            pltpu.VMEM((CHUNK,), jnp.int32),
            pltpu.VMEM((CHUNK,), jnp.int32),
            pltpu.VMEM((CHUNK, D), jnp.float32),
            pltpu.VMEM_SHARED((N_NODES, D), jnp.float32),
        ],
    )
    return f(feature, src, dst)


def _tc_body(f_ref, n0_ref, n1_ref, dp_ref, w1_ref, w2_ref, b_ref, o_ref):
    dsum = jnp.sum(dp_ref[...], axis=0)
    deg = jnp.maximum(dsum, 1.0)
    hkn = (n0_ref[...] + n1_ref[...]) * (1.0 / deg)
    acc = jnp.dot(f_ref[...], w1_ref[...], preferred_element_type=jnp.float32)
    acc = acc + jnp.dot(hkn, w2_ref[...], preferred_element_type=jnp.float32)
    o_ref[...] = jnp.maximum(acc + b_ref[...], 0.0)


BLK = 1000


def _tc_dense(feature, n0, n1, dparts, w1t, w2t, brow):
    grid = (N_NODES // BLK,)
    return pl.pallas_call(
        _tc_body,
        grid=grid,
        in_specs=[
            pl.BlockSpec((BLK, D), lambda i: (i, 0)),
            pl.BlockSpec((BLK, D), lambda i: (i, 0)),
            pl.BlockSpec((BLK, D), lambda i: (i, 0)),
            pl.BlockSpec((NUM_TILES, BLK, 1), lambda i: (0, i, 0)),
            pl.BlockSpec((D, D), lambda i: (0, 0)),
            pl.BlockSpec((D, D), lambda i: (0, 0)),
            pl.BlockSpec((1, D), lambda i: (0, 0)),
        ],
        out_specs=pl.BlockSpec((BLK, D), lambda i: (i, 0)),
        out_shape=jax.ShapeDtypeStruct((N_NODES, D), jnp.float32),
    )(feature, n0, n1, dparts, w1t, w2t, brow)


def kernel(feature, edge_index, W, b):
    src = edge_index[0].astype(jnp.int32)
    dst = edge_index[1].astype(jnp.int32)
    nparts, dhist = _sc_aggregate(feature, src, dst)
    wt = W.T
    w1t = wt[:D]
    w2t = wt[D:]
    dparts = dhist.reshape(NUM_TILES, N_NODES, 1)
    brow = b.reshape(1, D)
    return _tc_dense(feature, nparts[0], nparts[1], dparts, w1t, w2t, brow)


# speedup vs baseline: 2.6866x; 2.6866x over previous
import functools

import jax
import jax.numpy as jnp
from jax import lax
from jax.experimental import pallas as pl
from jax.experimental.pallas import tpu as pltpu
from jax.experimental.pallas import tpu_sc as plsc

N_NODES = 10000
N_EDGES = 320000
D = 128
HALF = D // 2

NUM_CORES = 2
NUM_SUBCORES = 16
NUM_TILES = NUM_CORES * NUM_SUBCORES
EDGES_PER_TILE = N_EDGES // NUM_TILES
CHUNK = 80
NCHUNKS = EDGES_PER_TILE // CHUNK
ROWS_PER_TILE = 624
TAIL_BASE = ROWS_PER_TILE * NUM_SUBCORES
TAIL = N_NODES - TAIL_BASE


def _sc_body(with_deg, feat_h, src_h, dst_h, nout_h, *rest):
    if with_deg:
        dout_h, zbuf, degl, idx_s, idx_d, rows, nacc = rest
    else:
        dout_h = degl = None
        zbuf, idx_s, idx_d, rows, nacc = rest

    c = lax.axis_index("c")
    s = lax.axis_index("s")
    wid = c * NUM_SUBCORES + s

    z16 = jnp.zeros((16,), jnp.float32)

    def zero_zbuf(i, carry):
        for k in range(HALF // 16):
            zbuf[i, pl.ds(k * 16, 16)] = z16
        return carry

    lax.fori_loop(0, ROWS_PER_TILE, zero_zbuf, 0)

    if with_deg:
        def zero_deg(i, carry):
            degl[pl.ds(i * 16, 16)] = z16
            return carry

        lax.fori_loop(0, N_NODES // 16, zero_deg, 0)

    pltpu.sync_copy(zbuf, nacc.at[pl.ds(s * ROWS_PER_TILE, ROWS_PER_TILE)])

    @pl.when(s == NUM_SUBCORES - 1)
    def _():
        pltpu.sync_copy(zbuf.at[pl.ds(0, TAIL)], nacc.at[pl.ds(TAIL_BASE, TAIL)])

    plsc.subcore_barrier()

    base = wid * EDGES_PER_TILE
    ones16 = jnp.ones((16,), jnp.float32)

    def step(j, carry):
        off = base + j * CHUNK
        pltpu.sync_copy(src_h.at[pl.ds(off, CHUNK)], idx_s)
        pltpu.sync_copy(dst_h.at[pl.ds(off, CHUNK)], idx_d)
        pltpu.sync_copy(feat_h.at[idx_s], rows)
        pltpu.sync_copy(rows, nacc.at[idx_d], add=True)
        if with_deg:
            for k in range(CHUNK // 16):
                i16 = idx_d[pl.ds(k * 16, 16)]
                plsc.addupdate_scatter(degl, [i16], ones16)
        return carry

    lax.fori_loop(0, NCHUNKS, step, 0)
    plsc.subcore_barrier()

    pltpu.sync_copy(nacc.at[pl.ds(s * ROWS_PER_TILE, ROWS_PER_TILE)],
                    nout_h.at[c, pl.ds(s * ROWS_PER_TILE, ROWS_PER_TILE)])

    @pl.when(s == NUM_SUBCORES - 1)
    def _():
        pltpu.sync_copy(nacc.at[pl.ds(TAIL_BASE, TAIL)],
                        nout_h.at[c, pl.ds(TAIL_BASE, TAIL)])

    if with_deg:
        pltpu.sync_copy(degl, dout_h.at[wid, 0])


def _sc_half(feat_half, src, dst, with_deg):
    mesh = plsc.VectorSubcoreMesh(core_axis_name="c", subcore_axis_name="s")
    out_type = [jax.ShapeDtypeStruct((NUM_CORES, N_NODES, HALF), jnp.float32)]
    if with_deg:
        out_type.append(
            jax.ShapeDtypeStruct((NUM_TILES, 1, N_NODES), jnp.float32))
    scratch = [pltpu.VMEM((ROWS_PER_TILE, HALF), jnp.float32)]
    if with_deg:
        scratch.append(pltpu.VMEM((N_NODES,), jnp.float32))
    scratch += [
        pltpu.VMEM((CHUNK,), jnp.int32),
        pltpu.VMEM((CHUNK,), jnp.int32),
        pltpu.VMEM((CHUNK, HALF), jnp.float32),
        pltpu.VMEM_SHARED((N_NODES, HALF), jnp.float32),
    ]
    f = pl.kernel(
        functools.partial(_sc_body, with_deg),
        out_type=out_type,
        mesh=mesh,
        compiler_params=pltpu.CompilerParams(
            needs_layout_passes=False, use_tc_tiling_on_sc=False),
        scratch_types=scratch,
    )
    return f(feat_half, src, dst)


def _tc_body(f_ref, n0a_ref, n1a_ref, n0b_ref, n1b_ref, dp_ref,
             w1_ref, w2a_ref, w2b_ref, b_ref, o_ref):
    dsum = jnp.sum(dp_ref[...], axis=0)
    rec = 1.0 / jnp.maximum(dsum, 1.0)
    hkna = (n0a_ref[...] + n1a_ref[...]) * rec
    hknb = (n0b_ref[...] + n1b_ref[...]) * rec
    acc = jnp.dot(f_ref[...], w1_ref[...], preferred_element_type=jnp.float32)
    acc = acc + jnp.dot(hkna, w2a_ref[...], preferred_element_type=jnp.float32)
    acc = acc + jnp.dot(hknb, w2b_ref[...], preferred_element_type=jnp.float32)
    o_ref[...] = jnp.maximum(acc + b_ref[...], 0.0)


BLK = 1000


def _tc_dense(feature, na, nb, dparts, w1t, w2at, w2bt, brow):
    grid = (N_NODES // BLK,)
    return pl.pallas_call(
        _tc_body,
        grid=grid,
        in_specs=[
            pl.BlockSpec((BLK, D), lambda i: (i, 0)),
            pl.BlockSpec((BLK, HALF), lambda i: (i, 0)),
            pl.BlockSpec((BLK, HALF), lambda i: (i, 0)),
            pl.BlockSpec((BLK, HALF), lambda i: (i, 0)),
            pl.BlockSpec((BLK, HALF), lambda i: (i, 0)),
            pl.BlockSpec((NUM_TILES, BLK, 1), lambda i: (0, i, 0)),
            pl.BlockSpec((D, D), lambda i: (0, 0)),
            pl.BlockSpec((HALF, D), lambda i: (0, 0)),
            pl.BlockSpec((HALF, D), lambda i: (0, 0)),
            pl.BlockSpec((1, D), lambda i: (0, 0)),
        ],
        out_specs=pl.BlockSpec((BLK, D), lambda i: (i, 0)),
        out_shape=jax.ShapeDtypeStruct((N_NODES, D), jnp.float32),
    )(feature, na[0], na[1], nb[0], nb[1], dparts, w1t, w2at, w2bt, brow)


def kernel(feature, edge_index, W, b):
    src = edge_index[0].astype(jnp.int32)
    dst = edge_index[1].astype(jnp.int32)
    feat_a = feature[:, :HALF]
    feat_b = feature[:, HALF:]
    na, dhist = _sc_half(feat_a, src, dst, with_deg=True)
    (nb,) = _sc_half(feat_b, src, dst, with_deg=False)
    wt = W.T
    w1t = wt[:D]
    w2at = wt[D:D + HALF]
    w2bt = wt[D + HALF:]
    dparts = dhist.reshape(NUM_TILES, N_NODES, 1)
    brow = b.reshape(1, D)
    return _tc_dense(feature, na, nb, dparts, w1t, w2at, w2bt, brow)


# speedup vs baseline: 6.3941x; 2.3800x over previous
import jax
import jax.numpy as jnp
from jax import lax
from jax.experimental import pallas as pl
from jax.experimental.pallas import tpu as pltpu
from jax.experimental.pallas import tpu_sc as plsc

N_NODES = 10000
N_EDGES = 320000
D = 128
HALF = D // 2

NUM_CORES = 2
NUM_SUBCORES = 16
EDGES_PER_TILE = N_EDGES // NUM_SUBCORES
CHUNK = 80
NCHUNKS = EDGES_PER_TILE // CHUNK
ROWS_PER_TILE = 624
TAIL_BASE = ROWS_PER_TILE * NUM_SUBCORES
TAIL = N_NODES - TAIL_BASE
ZROWS = ROWS_PER_TILE // 4


def _sc_body(feat3_h, src2_h, dst2_h, nout_h, dout_h,
             zbuf, degl, idx_s, idx_d, rows0, rows1, nacc,
             g0, g1, sc0, sc1):
    c = lax.axis_index("c")
    s = lax.axis_index("s")

    z16 = jnp.zeros((16,), jnp.float32)
    ones16 = jnp.ones((16,), jnp.float32)

    def zero_zbuf(i, carry):
        for k in range(HALF // 16):
            zbuf[i, pl.ds(k * 16, 16)] = z16
        return carry

    lax.fori_loop(0, ZROWS, zero_zbuf, 0)

    def zero_deg(i, carry):
        degl[pl.ds(i * 16, 16)] = z16
        return carry

    lax.fori_loop(0, N_NODES // 16, zero_deg, 0)

    for q in range(4):
        pltpu.sync_copy(zbuf, nacc.at[pl.ds(s * ROWS_PER_TILE + q * ZROWS, ZROWS)])

    @pl.when(s == NUM_SUBCORES - 1)
    def _():
        pltpu.sync_copy(zbuf.at[pl.ds(0, TAIL)], nacc.at[pl.ds(TAIL_BASE, TAIL)])

    plsc.subcore_barrier()

    pltpu.sync_copy(src2_h.at[pl.ds(s * NCHUNKS, NCHUNKS)], idx_s)
    pltpu.sync_copy(dst2_h.at[pl.ds(s * NCHUNKS, NCHUNKS)], idx_d)

    feat_h = feat3_h.at[c]
    bufs = ((rows0, g0, sc0), (rows1, g1, sc1))

    def _gather(j, rows, g):
        return pltpu.make_async_copy(feat_h.at[idx_s.at[j]], rows, g)

    def _scatter(j, rows, sc):
        return pltpu.make_async_copy(rows, nacc.at[idx_d.at[j]], sc)

    def pipe(i, carry):
        for b in range(2):
            j = 2 * i + b
            rows, g, sc = bufs[b]
            rows_p, g_p, sc_p = bufs[1 - b]

            @pl.when(j < NCHUNKS)
            def _():
                @pl.when(j >= 2)
                def _():
                    _scatter(j - 2, rows, sc).wait()

                _gather(j, rows, g).start()

                @pl.when(c == 0)
                def _():
                    for k in range(CHUNK // 16):
                        i16 = idx_d[j, pl.ds(k * 16, 16)]
                        plsc.addupdate_scatter(degl, [i16], ones16)

            @pl.when(jnp.logical_and(j >= 1, j <= NCHUNKS))
            def _():
                p = j - 1
                _gather(p, rows_p, g_p).wait()
                _scatter(p, rows_p, sc_p).start(add=True)

        return carry

    lax.fori_loop(0, NCHUNKS // 2 + 1, pipe, 0)

    _scatter(NCHUNKS - 2, rows0, sc0).wait()
    _scatter(NCHUNKS - 1, rows1, sc1).wait()
    plsc.subcore_barrier()

    pltpu.sync_copy(nacc.at[pl.ds(s * ROWS_PER_TILE, ROWS_PER_TILE)],
                    nout_h.at[c, pl.ds(s * ROWS_PER_TILE, ROWS_PER_TILE)])

    @pl.when(s == NUM_SUBCORES - 1)
    def _():
        pltpu.sync_copy(nacc.at[pl.ds(TAIL_BASE, TAIL)],
                        nout_h.at[c, pl.ds(TAIL_BASE, TAIL)])

    @pl.when(c == 0)
    def _():
        pltpu.sync_copy(degl, dout_h.at[s, 0])


def _sc_aggregate(feat3, src2, dst2):
    mesh = plsc.VectorSubcoreMesh(core_axis_name="c", subcore_axis_name="s")
    f = pl.kernel(
        _sc_body,
        out_type=[
            jax.ShapeDtypeStruct((NUM_CORES, N_NODES, HALF), jnp.float32),
            jax.ShapeDtypeStruct((NUM_SUBCORES, 1, N_NODES), jnp.float32),
        ],
        mesh=mesh,
        compiler_params=pltpu.CompilerParams(
            needs_layout_passes=False, use_tc_tiling_on_sc=False),
        scratch_types=[
            pltpu.VMEM((ZROWS, HALF), jnp.float32),
            pltpu.VMEM((N_NODES,), jnp.float32),
            pltpu.VMEM((NCHUNKS, CHUNK), jnp.int32),
            pltpu.VMEM((NCHUNKS, CHUNK), jnp.int32),
            pltpu.VMEM((CHUNK, HALF), jnp.float32),
            pltpu.VMEM((CHUNK, HALF), jnp.float32),
            pltpu.VMEM_SHARED((N_NODES, HALF), jnp.float32),
            pltpu.SemaphoreType.DMA,
            pltpu.SemaphoreType.DMA,
            pltpu.SemaphoreType.DMA,
            pltpu.SemaphoreType.DMA,
        ],
    )
    return f(feat3, src2, dst2)


def _tc_body(f_ref, na_ref, nb_ref, dp_ref, w1_ref, w2a_ref, w2b_ref,
             b_ref, o_ref):
    dsum = jnp.sum(dp_ref[...], axis=0)
    rec = 1.0 / jnp.maximum(dsum, 1.0)
    hkna = na_ref[...] * rec
    hknb = nb_ref[...] * rec
    acc = jnp.dot(f_ref[...], w1_ref[...], preferred_element_type=jnp.float32)
    acc = acc + jnp.dot(hkna, w2a_ref[...], preferred_element_type=jnp.float32)
    acc = acc + jnp.dot(hknb, w2b_ref[...], preferred_element_type=jnp.float32)
    o_ref[...] = jnp.maximum(acc + b_ref[...], 0.0)


BLK = 1000


def _tc_dense(feature, na, nb, dparts, w1t, w2at, w2bt, brow):
    grid = (N_NODES // BLK,)
    return pl.pallas_call(
        _tc_body,
        grid=grid,
        in_specs=[
            pl.BlockSpec((BLK, D), lambda i: (i, 0)),
            pl.BlockSpec((BLK, HALF), lambda i: (i, 0)),
            pl.BlockSpec((BLK, HALF), lambda i: (i, 0)),
            pl.BlockSpec((NUM_SUBCORES, BLK, 1), lambda i: (0, i, 0)),
            pl.BlockSpec((D, D), lambda i: (0, 0)),
            pl.BlockSpec((HALF, D), lambda i: (0, 0)),
            pl.BlockSpec((HALF, D), lambda i: (0, 0)),
            pl.BlockSpec((1, D), lambda i: (0, 0)),
        ],
        out_specs=pl.BlockSpec((BLK, D), lambda i: (i, 0)),
        out_shape=jax.ShapeDtypeStruct((N_NODES, D), jnp.float32),
    )(feature, na, nb, dparts, w1t, w2at, w2bt, brow)


def kernel(feature, edge_index, W, b):
    src2 = edge_index[0].astype(jnp.int32).reshape(N_EDGES // CHUNK, CHUNK)
    dst2 = edge_index[1].astype(jnp.int32).reshape(N_EDGES // CHUNK, CHUNK)
    feat3 = jnp.stack([feature[:, :HALF], feature[:, HALF:]])
    nout, dhist = _sc_aggregate(feat3, src2, dst2)
    wt = W.T
    w1t = wt[:D]
    w2at = wt[D:D + HALF]
    w2bt = wt[D + HALF:]
    dparts = dhist.reshape(NUM_SUBCORES, N_NODES, 1)
    brow = b.reshape(1, D)
    return _tc_dense(feature, nout[0], nout[1], dparts, w1t, w2at, w2bt, brow)


# speedup vs baseline: 7.4661x; 1.1677x over previous
import jax
import jax.numpy as jnp
from jax import lax
from jax.experimental import pallas as pl
from jax.experimental.pallas import tpu as pltpu
from jax.experimental.pallas import tpu_sc as plsc

N_NODES = 10000
N_EDGES = 320000
D = 128
HALF = D // 2

NUM_CORES = 2
NUM_SUBCORES = 16
EDGES_PER_TILE = N_EDGES // NUM_SUBCORES
CHUNK = 80
NCHUNKS = EDGES_PER_TILE // CHUNK
ROWS_PER_TILE = 624
TAIL_BASE = ROWS_PER_TILE * NUM_SUBCORES
TAIL = N_NODES - TAIL_BASE
ZROWS = ROWS_PER_TILE // 4


NB = 4
GD = 2


def _sc_body(feat3_h, src2_h, dst2_h, nout_h, dout_h,
             zbuf, degl, idx_s, idx_d, r0, r1, r2, r3, nacc,
             g0, g1, g2, g3, s0, s1, s2, s3, isem):
    rowbufs = (r0, r1, r2, r3)
    gsems = (g0, g1, g2, g3)
    scsems = (s0, s1, s2, s3)
    c = lax.axis_index("c")
    s = lax.axis_index("s")

    z16 = jnp.zeros((16,), jnp.float32)
    ones16 = jnp.ones((16,), jnp.float32)

    ixs = pltpu.async_copy(src2_h.at[pl.ds(s * NCHUNKS, NCHUNKS)], idx_s, isem)
    ixd = pltpu.async_copy(dst2_h.at[pl.ds(s * NCHUNKS, NCHUNKS)], idx_d, isem)

    def zero_zbuf(i, carry):
        for k in range(HALF // 16):
            zbuf[i, pl.ds(k * 16, 16)] = z16
        return carry

    lax.fori_loop(0, ZROWS, zero_zbuf, 0)

    def zero_deg(i, carry):
        degl[pl.ds(i * 16, 16)] = z16
        return carry

    lax.fori_loop(0, N_NODES // 16, zero_deg, 0)

    for q in range(4):
        pltpu.sync_copy(zbuf, nacc.at[pl.ds(s * ROWS_PER_TILE + q * ZROWS, ZROWS)])

    @pl.when(s == NUM_SUBCORES - 1)
    def _():
        pltpu.sync_copy(zbuf.at[pl.ds(0, TAIL)], nacc.at[pl.ds(TAIL_BASE, TAIL)])

    ixs.wait()
    ixd.wait()
    plsc.subcore_barrier()

    feat_h = feat3_h.at[c]

    def _gather(j, b):
        return pltpu.make_async_copy(feat_h.at[idx_s.at[j]], rowbufs[b],
                                     gsems[b])

    def _scatter(j, b):
        return pltpu.make_async_copy(rowbufs[b], nacc.at[idx_d.at[j]],
                                     scsems[b])

    def pipe(i, carry):
        for b in range(NB):
            j = NB * i + b

            @pl.when(j < NCHUNKS)
            def _():
                @pl.when(j >= NB)
                def _():
                    _scatter(j - NB, b).wait()

                _gather(j, b).start()

                @pl.when(c == 0)
                def _():
                    for k in range(CHUNK // 16):
                        i16 = idx_d[j, pl.ds(k * 16, 16)]
                        plsc.addupdate_scatter(degl, [i16], ones16)

            pb = (b - GD) % NB

            @pl.when(jnp.logical_and(j >= GD, j < NCHUNKS + GD))
            def _():
                p = j - GD
                _gather(p, pb).wait()
                _scatter(p, pb).start(add=True)

        return carry

    lax.fori_loop(0, (NCHUNKS + GD + NB - 1) // NB, pipe, 0)

    for t in range(NB):
        q = NCHUNKS - NB + t
        _scatter(q, q % NB).wait()
    plsc.subcore_barrier()

    pltpu.sync_copy(nacc.at[pl.ds(s * ROWS_PER_TILE, ROWS_PER_TILE)],
                    nout_h.at[c, pl.ds(s * ROWS_PER_TILE, ROWS_PER_TILE)])

    @pl.when(s == NUM_SUBCORES - 1)
    def _():
        pltpu.sync_copy(nacc.at[pl.ds(TAIL_BASE, TAIL)],
                        nout_h.at[c, pl.ds(TAIL_BASE, TAIL)])

    @pl.when(c == 0)
    def _():
        pltpu.sync_copy(degl, dout_h.at[s, 0])


def _sc_aggregate(feat3, src2, dst2):
    mesh = plsc.VectorSubcoreMesh(core_axis_name="c", subcore_axis_name="s")
    f = pl.kernel(
        _sc_body,
        out_type=[
            jax.ShapeDtypeStruct((NUM_CORES, N_NODES, HALF), jnp.float32),
            jax.ShapeDtypeStruct((NUM_SUBCORES, 1, N_NODES), jnp.float32),
        ],
        mesh=mesh,
        compiler_params=pltpu.CompilerParams(
            needs_layout_passes=False, use_tc_tiling_on_sc=False),
        scratch_types=[
            pltpu.VMEM((ZROWS, HALF), jnp.float32),
            pltpu.VMEM((N_NODES,), jnp.float32),
            pltpu.VMEM((NCHUNKS, CHUNK), jnp.int32),
            pltpu.VMEM((NCHUNKS, CHUNK), jnp.int32),
        ] + [pltpu.VMEM((CHUNK, HALF), jnp.float32) for _ in range(NB)]
        + [pltpu.VMEM_SHARED((N_NODES, HALF), jnp.float32)]
        + [pltpu.SemaphoreType.DMA for _ in range(2 * NB + 1)],
    )
    return f(feat3, src2, dst2)


def _tc_body(f_ref, na_ref, nb_ref, dp_ref, w1_ref, w2a_ref, w2b_ref,
             b_ref, o_ref):
    dsum = jnp.sum(dp_ref[...], axis=0)
    rec = 1.0 / jnp.maximum(dsum, 1.0)
    hkna = na_ref[...] * rec
    hknb = nb_ref[...] * rec
    acc = jnp.dot(f_ref[...], w1_ref[...], preferred_element_type=jnp.float32)
    acc = acc + jnp.dot(hkna, w2a_ref[...], preferred_element_type=jnp.float32)
    acc = acc + jnp.dot(hknb, w2b_ref[...], preferred_element_type=jnp.float32)
    o_ref[...] = jnp.maximum(acc + b_ref[...], 0.0)


BLK = 1000


def _tc_dense(feature, na, nb, dparts, w1t, w2at, w2bt, brow):
    grid = (N_NODES // BLK,)
    return pl.pallas_call(
        _tc_body,
        grid=grid,
        in_specs=[
            pl.BlockSpec((BLK, D), lambda i: (i, 0)),
            pl.BlockSpec((BLK, HALF), lambda i: (i, 0)),
            pl.BlockSpec((BLK, HALF), lambda i: (i, 0)),
            pl.BlockSpec((NUM_SUBCORES, BLK, 1), lambda i: (0, i, 0)),
            pl.BlockSpec((D, D), lambda i: (0, 0)),
            pl.BlockSpec((HALF, D), lambda i: (0, 0)),
            pl.BlockSpec((HALF, D), lambda i: (0, 0)),
            pl.BlockSpec((1, D), lambda i: (0, 0)),
        ],
        out_specs=pl.BlockSpec((BLK, D), lambda i: (i, 0)),
        out_shape=jax.ShapeDtypeStruct((N_NODES, D), jnp.float32),
    )(feature, na, nb, dparts, w1t, w2at, w2bt, brow)


def kernel(feature, edge_index, W, b):
    src2 = edge_index[0].astype(jnp.int32).reshape(N_EDGES // CHUNK, CHUNK)
    dst2 = edge_index[1].astype(jnp.int32).reshape(N_EDGES // CHUNK, CHUNK)
    feat3 = jnp.stack([feature[:, :HALF], feature[:, HALF:]])
    nout, dhist = _sc_aggregate(feat3, src2, dst2)
    wt = W.T
    w1t = wt[:D]
    w2at = wt[D:D + HALF]
    w2bt = wt[D + HALF:]
    dparts = dhist.reshape(NUM_SUBCORES, N_NODES, 1)
    brow = b.reshape(1, D)
    return _tc_dense(feature, nout[0], nout[1], dparts, w1t, w2at, w2bt, brow)


# speedup vs baseline: 12.4779x; 1.6713x over previous
import jax
import jax.numpy as jnp
from jax import lax
from jax.experimental import pallas as pl
from jax.experimental.pallas import tpu as pltpu
from jax.experimental.pallas import tpu_sc as plsc

N_NODES = 10000
N_EDGES = 320000
D = 128
HALF = D // 2

NUM_CORES = 2
NUM_SUBCORES = 16
EDGES_PER_TILE = N_EDGES // NUM_SUBCORES
CHUNK = 80
NCHUNKS = EDGES_PER_TILE // CHUNK
ROWS_PER_TILE = 624
TAIL_BASE = ROWS_PER_TILE * NUM_SUBCORES
TAIL = N_NODES - TAIL_BASE
ZROWS = ROWS_PER_TILE // 4


NB = 4
GD = 2


def _sc_body(feat3_h, src2_h, dst2_h, nout_h, dout_h,
             zbuf, degl, idx_s, idx_d, r0, r1, r2, r3, nacc,
             g0, g1, g2, g3, s0, s1, s2, s3, isem):
    rowbufs = (r0, r1, r2, r3)
    gsems = (g0, g1, g2, g3)
    scsems = (s0, s1, s2, s3)
    c = lax.axis_index("c")
    s = lax.axis_index("s")

    z16 = jnp.zeros((16,), jnp.float32)
    ones16 = jnp.ones((16,), jnp.float32)

    ixs = pltpu.async_copy(src2_h.at[pl.ds(s * NCHUNKS, NCHUNKS)], idx_s, isem)
    ixd = pltpu.async_copy(dst2_h.at[pl.ds(s * NCHUNKS, NCHUNKS)], idx_d, isem)

    def zero_zbuf(i, carry):
        for k in range(HALF // 16):
            zbuf[i, pl.ds(k * 16, 16)] = z16
        return carry

    lax.fori_loop(0, ZROWS, zero_zbuf, 0)

    def zero_deg(i, carry):
        degl[pl.ds(i * 16, 16)] = z16
        return carry

    lax.fori_loop(0, N_NODES // 16, zero_deg, 0)

    for q in range(4):
        pltpu.sync_copy(zbuf, nacc.at[pl.ds(s * ROWS_PER_TILE + q * ZROWS, ZROWS)])

    @pl.when(s == NUM_SUBCORES - 1)
    def _():
        pltpu.sync_copy(zbuf.at[pl.ds(0, TAIL)], nacc.at[pl.ds(TAIL_BASE, TAIL)])

    ixs.wait()
    ixd.wait()
    plsc.subcore_barrier()

    feat_h = feat3_h.at[c]

    def _gather(j, b):
        return pltpu.make_async_copy(feat_h.at[idx_s.at[j]], rowbufs[b],
                                     gsems[b])

    def _scatter(j, b):
        return pltpu.make_async_copy(rowbufs[b], nacc.at[idx_d.at[j]],
                                     scsems[b])

    def pipe(i, carry):
        for b in range(NB):
            j = NB * i + b

            @pl.when(j < NCHUNKS)
            def _():
                @pl.when(j >= NB)
                def _():
                    _scatter(j - NB, b).wait()

                _gather(j, b).start()

                @pl.when(c == 0)
                def _():
                    for k in range(CHUNK // 16):
                        i16 = idx_d[j, pl.ds(k * 16, 16)]
                        plsc.addupdate_scatter(degl, [i16], ones16)

            pb = (b - GD) % NB

            @pl.when(jnp.logical_and(j >= GD, j < NCHUNKS + GD))
            def _():
                p = j - GD
                _gather(p, pb).wait()
                _scatter(p, pb).start(add=True)

        return carry

    lax.fori_loop(0, (NCHUNKS + GD + NB - 1) // NB, pipe, 0)

    for t in range(NB):
        q = NCHUNKS - NB + t
        _scatter(q, q % NB).wait()
    plsc.subcore_barrier()

    pltpu.sync_copy(nacc.at[pl.ds(s * ROWS_PER_TILE, ROWS_PER_TILE)],
                    nout_h.at[c, pl.ds(s * ROWS_PER_TILE, ROWS_PER_TILE)])

    @pl.when(s == NUM_SUBCORES - 1)
    def _():
        pltpu.sync_copy(nacc.at[pl.ds(TAIL_BASE, TAIL)],
                        nout_h.at[c, pl.ds(TAIL_BASE, TAIL)])

    @pl.when(c == 0)
    def _():
        pltpu.sync_copy(degl, dout_h.at[s, 0])


def _sc_aggregate(feat3, src2, dst2):
    mesh = plsc.VectorSubcoreMesh(core_axis_name="c", subcore_axis_name="s")
    f = pl.kernel(
        _sc_body,
        out_type=[
            jax.ShapeDtypeStruct((NUM_CORES, N_NODES, HALF), jnp.float32),
            jax.ShapeDtypeStruct((NUM_SUBCORES, 1, N_NODES), jnp.float32),
        ],
        mesh=mesh,
        compiler_params=pltpu.CompilerParams(
            needs_layout_passes=False, use_tc_tiling_on_sc=False),
        scratch_types=[
            pltpu.VMEM((ZROWS, HALF), jnp.float32),
            pltpu.VMEM((N_NODES,), jnp.float32),
            pltpu.VMEM((NCHUNKS, CHUNK), jnp.int32),
            pltpu.VMEM((NCHUNKS, CHUNK), jnp.int32),
        ] + [pltpu.VMEM((CHUNK, HALF), jnp.float32) for _ in range(NB)]
        + [pltpu.VMEM_SHARED((N_NODES, HALF), jnp.float32)]
        + [pltpu.SemaphoreType.DMA for _ in range(2 * NB + 1)],
    )
    return f(feat3, src2, dst2)


def _tc_body(f_ref, na_ref, nb_ref, dp_ref, w1_ref, w2a_ref, w2b_ref,
             b_ref, o_ref):
    rec = 1.0 / jnp.maximum(dp_ref[...], 1.0)
    hkna = na_ref[...] * rec
    hknb = nb_ref[...] * rec
    acc = jnp.dot(f_ref[...], w1_ref[...], preferred_element_type=jnp.float32)
    acc = acc + jnp.dot(hkna, w2a_ref[...], preferred_element_type=jnp.float32)
    acc = acc + jnp.dot(hknb, w2b_ref[...], preferred_element_type=jnp.float32)
    o_ref[...] = jnp.maximum(acc + b_ref[...], 0.0)


BLK = 1000


def _tc_dense(feature, na, nb, dparts, w1t, w2at, w2bt, brow):
    grid = (N_NODES // BLK,)
    return pl.pallas_call(
        _tc_body,
        grid=grid,
        in_specs=[
            pl.BlockSpec((BLK, D), lambda i: (i, 0)),
            pl.BlockSpec((BLK, HALF), lambda i: (i, 0)),
            pl.BlockSpec((BLK, HALF), lambda i: (i, 0)),
            pl.BlockSpec((BLK, 1), lambda i: (i, 0)),
            pl.BlockSpec((D, D), lambda i: (0, 0)),
            pl.BlockSpec((HALF, D), lambda i: (0, 0)),
            pl.BlockSpec((HALF, D), lambda i: (0, 0)),
            pl.BlockSpec((1, D), lambda i: (0, 0)),
        ],
        out_specs=pl.BlockSpec((BLK, D), lambda i: (i, 0)),
        out_shape=jax.ShapeDtypeStruct((N_NODES, D), jnp.float32),
    )(feature, na, nb, dparts, w1t, w2at, w2bt, brow)


def kernel(feature, edge_index, W, b):
    src2 = edge_index[0].astype(jnp.int32).reshape(N_EDGES // CHUNK, CHUNK)
    dst2 = edge_index[1].astype(jnp.int32).reshape(N_EDGES // CHUNK, CHUNK)
    feat3 = jnp.stack([feature[:, :HALF], feature[:, HALF:]])
    nout, dhist = _sc_aggregate(feat3, src2, dst2)
    wt = W.T
    w1t = wt[:D]
    w2at = wt[D:D + HALF]
    w2bt = wt[D + HALF:]
    dparts = dhist.reshape(NUM_SUBCORES, N_NODES).sum(axis=0).reshape(N_NODES, 1)
    brow = b.reshape(1, D)
    return _tc_dense(feature, nout[0], nout[1], dparts, w1t, w2at, w2bt, brow)


# speedup vs baseline: 13.7347x; 1.1007x over previous
import jax
import jax.numpy as jnp
from jax import lax
from jax.experimental import pallas as pl
from jax.experimental.pallas import tpu as pltpu
from jax.experimental.pallas import tpu_sc as plsc

N_NODES = 10000
N_EDGES = 320000
D = 128
HALF = D // 2

NUM_CORES = 2
NUM_SUBCORES = 16
EDGES_PER_TILE = N_EDGES // NUM_SUBCORES
CHUNK = 80
NCHUNKS = EDGES_PER_TILE // CHUNK
ROWS_PER_TILE = 624
TAIL_BASE = ROWS_PER_TILE * NUM_SUBCORES
TAIL = N_NODES - TAIL_BASE
ZROWS = ROWS_PER_TILE // 4


NB = 4
GD = 2


def _sc_body(feat3_h, src2_h, dst2_h, nout_h, dout_h,
             zbuf, degl, idx_s, idx_d, r0, r1, r2, r3, nacc,
             g0, g1, g2, g3, s0, s1, s2, s3, isem):
    rowbufs = (r0, r1, r2, r3)
    gsems = (g0, g1, g2, g3)
    scsems = (s0, s1, s2, s3)
    c = lax.axis_index("c")
    s = lax.axis_index("s")

    z16 = jnp.zeros((16,), jnp.float32)
    ones16 = jnp.ones((16,), jnp.float32)

    ixs = pltpu.async_copy(src2_h.at[pl.ds(s * NCHUNKS, NCHUNKS)], idx_s, isem)
    ixd = pltpu.async_copy(dst2_h.at[pl.ds(s * NCHUNKS, NCHUNKS)], idx_d, isem)

    def zero_zbuf(i, carry):
        for k in range(HALF // 16):
            zbuf[i, pl.ds(k * 16, 16)] = z16
        return carry

    lax.fori_loop(0, ZROWS, zero_zbuf, 0)

    def zero_deg(i, carry):
        degl[pl.ds(i * 16, 16)] = z16
        return carry

    lax.fori_loop(0, N_NODES // 16, zero_deg, 0)

    for q in range(4):
        pltpu.sync_copy(zbuf, nacc.at[pl.ds(s * ROWS_PER_TILE + q * ZROWS, ZROWS)])

    @pl.when(s == NUM_SUBCORES - 1)
    def _():
        pltpu.sync_copy(zbuf.at[pl.ds(0, TAIL)], nacc.at[pl.ds(TAIL_BASE, TAIL)])

    ixs.wait()
    ixd.wait()
    plsc.subcore_barrier()

    feat_c = feat3_h.at[c]

    def _gather(j, b):
        return pltpu.make_async_copy(feat_c.at[idx_s.at[j]], rowbufs[b],
                                     gsems[b])

    def _scatter(j, b):
        return pltpu.make_async_copy(rowbufs[b], nacc.at[idx_d.at[j]],
                                     scsems[b])

    def pipe(i, carry):
        for b in range(NB):
            j = NB * i + b

            @pl.when(j < NCHUNKS)
            def _():
                @pl.when(j >= NB)
                def _():
                    _scatter(j - NB, b).wait()

                _gather(j, b).start()

                @pl.when(c == 0)
                def _():
                    for k in range(CHUNK // 16):
                        i16 = idx_d[j, pl.ds(k * 16, 16)]
                        plsc.addupdate_scatter(degl, [i16], ones16)

            pb = (b - GD) % NB

            @pl.when(jnp.logical_and(j >= GD, j < NCHUNKS + GD))
            def _():
                p = j - GD
                _gather(p, pb).wait()
                _scatter(p, pb).start(add=True)

        return carry

    lax.fori_loop(0, (NCHUNKS + GD + NB - 1) // NB, pipe, 0)

    for t in range(NB):
        q = NCHUNKS - NB + t
        _scatter(q, q % NB).wait()
    plsc.subcore_barrier()

    nout_c = nout_h.at[:, pl.ds(c * HALF, HALF)]
    pltpu.sync_copy(nacc.at[pl.ds(s * ROWS_PER_TILE, ROWS_PER_TILE)],
                    nout_c.at[pl.ds(s * ROWS_PER_TILE, ROWS_PER_TILE)])

    @pl.when(s == NUM_SUBCORES - 1)
    def _():
        pltpu.sync_copy(nacc.at[pl.ds(TAIL_BASE, TAIL)],
                        nout_c.at[pl.ds(TAIL_BASE, TAIL)])

    @pl.when(c == 0)
    def _():
        pltpu.sync_copy(degl, dout_h.at[s, 0])


def _sc_aggregate(feat3, src2, dst2):
    mesh = plsc.VectorSubcoreMesh(core_axis_name="c", subcore_axis_name="s")
    f = pl.kernel(
        _sc_body,
        out_type=[
            jax.ShapeDtypeStruct((N_NODES, D), jnp.float32),
            jax.ShapeDtypeStruct((NUM_SUBCORES, 1, N_NODES), jnp.float32),
        ],
        mesh=mesh,
        compiler_params=pltpu.CompilerParams(
            needs_layout_passes=False, use_tc_tiling_on_sc=False),
        scratch_types=[
            pltpu.VMEM((ZROWS, HALF), jnp.float32),
            pltpu.VMEM((N_NODES,), jnp.float32),
            pltpu.VMEM((NCHUNKS, CHUNK), jnp.int32),
            pltpu.VMEM((NCHUNKS, CHUNK), jnp.int32),
        ] + [pltpu.VMEM((CHUNK, HALF), jnp.float32) for _ in range(NB)]
        + [pltpu.VMEM_SHARED((N_NODES, HALF), jnp.float32)]
        + [pltpu.SemaphoreType.DMA for _ in range(2 * NB + 1)],
    )
    return f(feat3, src2, dst2)


def _tc_body(f_ref, n_ref, dp_ref, w_ref, b_ref, o_ref):
    rec = 1.0 / jnp.maximum(dp_ref[...], 1.0)
    hk = n_ref[...] * rec
    w = w_ref[...]
    dn = (((1,), (1,)), ((), ()))
    acc = lax.dot_general(f_ref[...], w[:, :D], dn,
                          preferred_element_type=jnp.float32)
    acc = acc + lax.dot_general(hk, w[:, D:], dn,
                                preferred_element_type=jnp.float32)
    o_ref[...] = jnp.maximum(acc + b_ref[...], 0.0)


BLK = 1000


def _tc_dense(feature, nsum, dcol, W, brow):
    grid = (N_NODES // BLK,)
    return pl.pallas_call(
        _tc_body,
        grid=grid,
        in_specs=[
            pl.BlockSpec((BLK, D), lambda i: (i, 0)),
            pl.BlockSpec((BLK, D), lambda i: (i, 0)),
            pl.BlockSpec((BLK, 1), lambda i: (i, 0)),
            pl.BlockSpec((D, 2 * D), lambda i: (0, 0)),
            pl.BlockSpec((1, D), lambda i: (0, 0)),
        ],
        out_specs=pl.BlockSpec((BLK, D), lambda i: (i, 0)),
        out_shape=jax.ShapeDtypeStruct((N_NODES, D), jnp.float32),
    )(feature, nsum, dcol, W, brow)


def kernel(feature, edge_index, W, b):
    src2 = edge_index[0].astype(jnp.int32).reshape(N_EDGES // CHUNK, CHUNK)
    dst2 = edge_index[1].astype(jnp.int32).reshape(N_EDGES // CHUNK, CHUNK)
    feat3 = jnp.stack([feature[:, :HALF], feature[:, HALF:]])
    nsum, dhist = _sc_aggregate(feat3, src2, dst2)
    dcol = dhist.reshape(NUM_SUBCORES, N_NODES).sum(axis=0).reshape(N_NODES, 1)
    return _tc_dense(feature, nsum, dcol, W, b.reshape(1, D))


# speedup vs baseline: 14.6969x; 1.0701x over previous
import jax
import jax.numpy as jnp
from jax import lax
from jax.experimental import pallas as pl
from jax.experimental.pallas import tpu as pltpu
from jax.experimental.pallas import tpu_sc as plsc

N_NODES = 10000
N_EDGES = 320000
D = 128
HALF = D // 2

NUM_CORES = 2
NUM_SUBCORES = 16
EDGES_PER_TILE = N_EDGES // NUM_SUBCORES
CHUNK = 80
NCHUNKS = EDGES_PER_TILE // CHUNK
ROWS_PER_TILE = 624
TAIL_BASE = ROWS_PER_TILE * NUM_SUBCORES
TAIL = N_NODES - TAIL_BASE
ZROWS = ROWS_PER_TILE // 4


NB = 6
GD = 3


def _sc_body(feat3_h, src2_h, dst2_h, nout_h, dout_h,
             zbuf, degl, idx_s, idx_d, r0, r1, r2, r3, r4, r5, nacc,
             g0, g1, g2, g3, g4, g5, s0, s1, s2, s3, s4, s5, isem):
    rowbufs = (r0, r1, r2, r3, r4, r5)
    gsems = (g0, g1, g2, g3, g4, g5)
    scsems = (s0, s1, s2, s3, s4, s5)
    c = lax.axis_index("c")
    s = lax.axis_index("s")

    z16 = jnp.zeros((16,), jnp.float32)
    ones16 = jnp.ones((16,), jnp.float32)

    ixs = pltpu.async_copy(src2_h.at[pl.ds(s * NCHUNKS, NCHUNKS)], idx_s, isem)
    ixd = pltpu.async_copy(dst2_h.at[pl.ds(s * NCHUNKS, NCHUNKS)], idx_d, isem)

    def zero_zbuf(i, carry):
        for k in range(HALF // 16):
            zbuf[i, pl.ds(k * 16, 16)] = z16
        return carry

    lax.fori_loop(0, ZROWS, zero_zbuf, 0)

    def zero_deg(i, carry):
        degl[pl.ds(i * 16, 16)] = z16
        return carry

    lax.fori_loop(0, N_NODES // 16, zero_deg, 0)

    for q in range(4):
        pltpu.sync_copy(zbuf, nacc.at[pl.ds(s * ROWS_PER_TILE + q * ZROWS, ZROWS)])

    @pl.when(s == NUM_SUBCORES - 1)
    def _():
        pltpu.sync_copy(zbuf.at[pl.ds(0, TAIL)], nacc.at[pl.ds(TAIL_BASE, TAIL)])

    ixs.wait()
    ixd.wait()
    plsc.subcore_barrier()

    feat_c = feat3_h.at[c]

    def _gather(j, b):
        return pltpu.make_async_copy(feat_c.at[idx_s.at[j]], rowbufs[b],
                                     gsems[b])

    def _scatter(j, b):
        return pltpu.make_async_copy(rowbufs[b], nacc.at[idx_d.at[j]],
                                     scsems[b])

    def pipe(i, carry):
        for b in range(NB):
            j = NB * i + b

            @pl.when(j < NCHUNKS)
            def _():
                @pl.when(j >= NB)
                def _():
                    _scatter(j - NB, b).wait()

                _gather(j, b).start()

                @pl.when(c == 0)
                def _():
                    for k in range(CHUNK // 16):
                        i16 = idx_d[j, pl.ds(k * 16, 16)]
                        plsc.addupdate_scatter(degl, [i16], ones16)

            pb = (b - GD) % NB

            @pl.when(jnp.logical_and(j >= GD, j < NCHUNKS + GD))
            def _():
                p = j - GD
                _gather(p, pb).wait()
                _scatter(p, pb).start(add=True)

        return carry

    lax.fori_loop(0, (NCHUNKS + GD + NB - 1) // NB, pipe, 0)

    for t in range(NB):
        q = NCHUNKS - NB + t
        _scatter(q, q % NB).wait()
    plsc.subcore_barrier()

    nout_c = nout_h.at[:, pl.ds(c * HALF, HALF)]
    pltpu.sync_copy(nacc.at[pl.ds(s * ROWS_PER_TILE, ROWS_PER_TILE)],
                    nout_c.at[pl.ds(s * ROWS_PER_TILE, ROWS_PER_TILE)])

    @pl.when(s == NUM_SUBCORES - 1)
    def _():
        pltpu.sync_copy(nacc.at[pl.ds(TAIL_BASE, TAIL)],
                        nout_c.at[pl.ds(TAIL_BASE, TAIL)])

    @pl.when(c == 0)
    def _():
        pltpu.sync_copy(degl, dout_h.at[s, 0])


def _sc_aggregate(feat3, src2, dst2):
    mesh = plsc.VectorSubcoreMesh(core_axis_name="c", subcore_axis_name="s")
    f = pl.kernel(
        _sc_body,
        out_type=[
            jax.ShapeDtypeStruct((N_NODES, D), jnp.float32),
            jax.ShapeDtypeStruct((NUM_SUBCORES, 1, N_NODES), jnp.float32),
        ],
        mesh=mesh,
        compiler_params=pltpu.CompilerParams(
            needs_layout_passes=False, use_tc_tiling_on_sc=False),
        scratch_types=[
            pltpu.VMEM((ZROWS, HALF), jnp.float32),
            pltpu.VMEM((N_NODES,), jnp.float32),
            pltpu.VMEM((NCHUNKS, CHUNK), jnp.int32),
            pltpu.VMEM((NCHUNKS, CHUNK), jnp.int32),
        ] + [pltpu.VMEM((CHUNK, HALF), jnp.float32) for _ in range(NB)]
        + [pltpu.VMEM_SHARED((N_NODES, HALF), jnp.float32)]
        + [pltpu.SemaphoreType.DMA for _ in range(2 * NB + 1)],
    )
    return f(feat3, src2, dst2)


def _tc_body(f_ref, n_ref, dp_ref, w_ref, b_ref, o_ref):
    rec = 1.0 / jnp.maximum(dp_ref[...], 1.0)
    hk = n_ref[...] * rec
    w = w_ref[...]
    dn = (((1,), (1,)), ((), ()))
    acc = lax.dot_general(f_ref[...], w[:, :D], dn,
                          preferred_element_type=jnp.float32)
    acc = acc + lax.dot_general(hk, w[:, D:], dn,
                                preferred_element_type=jnp.float32)
    o_ref[...] = jnp.maximum(acc + b_ref[...], 0.0)


BLK = 1000


def _tc_dense(feature, nsum, dcol, W, brow):
    grid = (N_NODES // BLK,)
    return pl.pallas_call(
        _tc_body,
        grid=grid,
        in_specs=[
            pl.BlockSpec((BLK, D), lambda i: (i, 0)),
            pl.BlockSpec((BLK, D), lambda i: (i, 0)),
            pl.BlockSpec((BLK, 1), lambda i: (i, 0)),
            pl.BlockSpec((D, 2 * D), lambda i: (0, 0)),
            pl.BlockSpec((1, D), lambda i: (0, 0)),
        ],
        out_specs=pl.BlockSpec((BLK, D), lambda i: (i, 0)),
        out_shape=jax.ShapeDtypeStruct((N_NODES, D), jnp.float32),
    )(feature, nsum, dcol, W, brow)


def kernel(feature, edge_index, W, b):
    src2 = edge_index[0].astype(jnp.int32).reshape(N_EDGES // CHUNK, CHUNK)
    dst2 = edge_index[1].astype(jnp.int32).reshape(N_EDGES // CHUNK, CHUNK)
    feat3 = jnp.stack([feature[:, :HALF], feature[:, HALF:]])
    nsum, dhist = _sc_aggregate(feat3, src2, dst2)
    dcol = dhist.reshape(NUM_SUBCORES, N_NODES).sum(axis=0).reshape(N_NODES, 1)
    return _tc_dense(feature, nsum, dcol, W, b.reshape(1, D))
